# baseline (speedup 1.0000x reference)
import functools

import jax
import jax.numpy as jnp
from jax import lax
from jax.experimental import pallas as pl
from jax.experimental.pallas import tpu as pltpu
from jax.experimental.pallas import tpu_sc as plsc

D = 64
NC, NS = 2, 16
NW = NC * NS


@functools.partial(jax.jit, static_argnums=(2, 3))
def _lookup_sum(idx2, w, n, c):
    per_w = n // NW
    n_chunks = per_w // c
    assert n_chunks % 2 == 0
    c2 = 2 * c
    mesh = plsc.VectorSubcoreMesh(core_axis_name="c", subcore_axis_name="s")

    def body(idx_hbm, w_hbm, out_hbm,
             idx_v, b0, b1, o0, o1, g0, g1, wb0, wb1):
        wid = lax.axis_index("s") * NC + lax.axis_index("c")
        base = wid * per_w
        chunk0 = wid * n_chunks
        buf = (b0, b1)
        obuf = (o0, o1)
        g = (g0, g1)
        wb = (wb0, wb1)

        pltpu.sync_copy(idx_hbm.at[pl.ds(chunk0, n_chunks), :], idx_v)

        def fire_gather(k, b):
            pltpu.async_copy(w_hbm.at[idx_v.at[k]], buf[b], g[b])

        def wait_gather(k, b):
            pltpu.make_async_copy(w_hbm.at[idx_v.at[k]], buf[b], g[b]).wait()

        def wait_wb(k, b):
            pltpu.make_async_copy(
                obuf[b], out_hbm.at[pl.ds(base + k * c, c)], wb[b]).wait()

        fire_gather(0, 0)

        def pair(k2, carry):
            for b in (0, 1):
                k = 2 * k2 + b
                b1 = 1 - b

                @pl.when(k + 1 < n_chunks)
                def _():
                    fire_gather(k + 1, b1)

                @pl.when(k >= 2)
                def _():
                    wait_wb(k - 2, b)

                wait_gather(k, b)

                def add_row(i, carry2):
                    for j in range(D // 16):
                        s = pl.ds(j * 16, 16)
                        obuf[b][i, s] = buf[b][2 * i, s] + buf[b][2 * i + 1, s]
                    return carry2

                lax.fori_loop(0, c, add_row, 0, unroll=4)

                pltpu.async_copy(
                    obuf[b], out_hbm.at[pl.ds(base + k * c, c)], wb[b])
            return carry

        lax.fori_loop(0, n_chunks // 2, pair, 0)
        wait_wb(n_chunks - 2, 0)
        wait_wb(n_chunks - 1, 1)

    return pl.kernel(
        body,
        out_type=jax.ShapeDtypeStruct((n, D), jnp.float32),
        mesh=mesh,
        compiler_params=pltpu.CompilerParams(use_tc_tiling_on_sc=False),
        scratch_types=[
            pltpu.VMEM((n_chunks, c2), jnp.int32),
            pltpu.VMEM((c2, D), jnp.float32),
            pltpu.VMEM((c2, D), jnp.float32),
            pltpu.VMEM((c, D), jnp.float32),
            pltpu.VMEM((c, D), jnp.float32),
            pltpu.SemaphoreType.DMA,
            pltpu.SemaphoreType.DMA,
            pltpu.SemaphoreType.DMA,
            pltpu.SemaphoreType.DMA,
        ],
    )(idx2, w)


def kernel(x_coord, Wx, Wy):
    b, l, _ = x_coord.shape
    n = b * l
    c = 128
    grid = Wx.shape[0]
    w = jnp.concatenate([Wx, Wy], axis=0)
    idx2 = (x_coord + jnp.array([0, grid], jnp.int32)).reshape(n // c, 2 * c)
    out = _lookup_sum(idx2, w, n, c)
    return out.reshape(b, l, D)


# speedup vs baseline: 1.0195x; 1.0195x over previous
import functools

import jax
import jax.numpy as jnp
from jax import lax
from jax.experimental import pallas as pl
from jax.experimental.pallas import tpu as pltpu
from jax.experimental.pallas import tpu_sc as plsc

D = 64
NC, NS = 2, 16
NW = NC * NS


@functools.partial(jax.jit, static_argnums=(3, 4))
def _lookup_sum(idx2, wx, wy, n, c):
    per_w = n // NW
    n_chunks = per_w // c
    assert n_chunks % 2 == 0
    c2 = 2 * c
    mesh = plsc.VectorSubcoreMesh(core_axis_name="c", subcore_axis_name="s")

    def body(idx_hbm, wx_hbm, wy_hbm, out_hbm,
             idx_v, ix0, ix1, iy0, iy1, ba0, ba1, bb0, bb1,
             ga0, ga1, gb0, gb1, wb0, wb1):
        wid = lax.axis_index("s") * NC + lax.axis_index("c")
        base = wid * per_w
        chunk0 = wid * n_chunks
        idxx = (ix0, ix1)
        idxy = (iy0, iy1)
        bufa = (ba0, ba1)
        bufb = (bb0, bb1)
        ga = (ga0, ga1)
        gb = (gb0, gb1)
        wb = (wb0, wb1)

        pltpu.sync_copy(idx_hbm.at[pl.ds(chunk0, n_chunks), :], idx_v)

        lanes = lax.iota(jnp.int32, 16)

        def stage_and_fire(k, b):
            row = idx_v.at[k]
            for j in range(c2 // 32):
                col = lanes * 2 + (32 * j)
                idxx[b][pl.ds(j * 16, 16)] = plsc.load_gather(row, [col])
                idxy[b][pl.ds(j * 16, 16)] = plsc.load_gather(row, [col + 1])
            pltpu.async_copy(wx_hbm.at[idxx[b]], bufa[b], ga[b])
            pltpu.async_copy(wy_hbm.at[idxy[b]], bufb[b], gb[b])

        def wait_gathers(b):
            pltpu.make_async_copy(wx_hbm.at[idxx[b]], bufa[b], ga[b]).wait()
            pltpu.make_async_copy(wy_hbm.at[idxy[b]], bufb[b], gb[b]).wait()

        def wait_wb(k, b):
            pltpu.make_async_copy(
                bufa[b], out_hbm.at[pl.ds(base + k * c, c)], wb[b]).wait()

        stage_and_fire(0, 0)

        def pair(k2, carry):
            for b in (0, 1):
                k = 2 * k2 + b
                b1 = 1 - b

                @pl.when(k >= 1)
                def _():
                    wait_wb(k - 1, b1)

                @pl.when(k + 1 < n_chunks)
                def _():
                    stage_and_fire(k + 1, b1)

                wait_gathers(b)

                def add_row(i, carry2):
                    for j in range(D // 16):
                        s = pl.ds(j * 16, 16)
                        bufa[b][i, s] = bufa[b][i, s] + bufb[b][i, s]
                    return carry2

                lax.fori_loop(0, c, add_row, 0, unroll=4)
                pltpu.async_copy(
                    bufa[b], out_hbm.at[pl.ds(base + k * c, c)], wb[b])
            return carry

        lax.fori_loop(0, n_chunks // 2, pair, 0)
        wait_wb(n_chunks - 1, 1)

    return pl.kernel(
        body,
        out_type=jax.ShapeDtypeStruct((n, D), jnp.float32),
        mesh=mesh,
        compiler_params=pltpu.CompilerParams(
            use_tc_tiling_on_sc=False, needs_layout_passes=False),
        scratch_types=[
            pltpu.VMEM((n_chunks, c2), jnp.int32),
            pltpu.VMEM((c,), jnp.int32),
            pltpu.VMEM((c,), jnp.int32),
            pltpu.VMEM((c,), jnp.int32),
            pltpu.VMEM((c,), jnp.int32),
            pltpu.VMEM((c, D), jnp.float32),
            pltpu.VMEM((c, D), jnp.float32),
            pltpu.VMEM((c, D), jnp.float32),
            pltpu.VMEM((c, D), jnp.float32),
            pltpu.SemaphoreType.DMA,
            pltpu.SemaphoreType.DMA,
            pltpu.SemaphoreType.DMA,
            pltpu.SemaphoreType.DMA,
            pltpu.SemaphoreType.DMA,
            pltpu.SemaphoreType.DMA,
        ],
    )(idx2, wx, wy)


def kernel(x_coord, Wx, Wy):
    b, l, _ = x_coord.shape
    n = b * l
    c = 128
    idx2 = x_coord.reshape(n // c, 2 * c)
    out = _lookup_sum(idx2, Wx, Wy, n, c)
    return out.reshape(b, l, D)


# speedup vs baseline: 1.9116x; 1.8752x over previous
import functools

import jax
import jax.numpy as jnp
from jax import lax
from jax.experimental import pallas as pl
from jax.experimental.pallas import tpu as pltpu
from jax.experimental.pallas import tpu_sc as plsc

D = 64
NC, NS = 2, 16
NW = NC * NS


@functools.partial(jax.jit, static_argnums=(4, 5))
def _lookup_sum(ix, iy, wx, wy, n, c):
    per_w = n // NW
    n_chunks = per_w // c
    assert n_chunks % 2 == 0
    mesh = plsc.VectorSubcoreMesh(core_axis_name="c", subcore_axis_name="s")

    def body(ix_hbm, iy_hbm, wx_hbm, wy_hbm, out_hbm,
             idxx_v, idxy_v, ba0, ba1, bb0, bb1,
             ga0, ga1, gb0, gb1, wb0, wb1):
        wid = lax.axis_index("s") * NC + lax.axis_index("c")
        base = wid * per_w
        bufa = (ba0, ba1)
        bufb = (bb0, bb1)
        ga = (ga0, ga1)
        gb = (gb0, gb1)
        wb = (wb0, wb1)

        pltpu.sync_copy(ix_hbm.at[pl.ds(base, per_w)], idxx_v)
        pltpu.sync_copy(iy_hbm.at[pl.ds(base, per_w)], idxy_v)

        def fire_gathers(k, b):
            s = pl.ds(k * c, c)
            pltpu.async_copy(wx_hbm.at[idxx_v.at[s]], bufa[b], ga[b])
            pltpu.async_copy(wy_hbm.at[idxy_v.at[s]], bufb[b], gb[b])

        def wait_gathers(k, b):
            s = pl.ds(k * c, c)
            pltpu.make_async_copy(wx_hbm.at[idxx_v.at[s]], bufa[b], ga[b]).wait()
            pltpu.make_async_copy(wy_hbm.at[idxy_v.at[s]], bufb[b], gb[b]).wait()

        def wait_wb(k, b):
            pltpu.make_async_copy(
                bufa[b], out_hbm.at[pl.ds(base + k * c, c)], wb[b]).wait()

        fire_gathers(0, 0)

        def pair(k2, carry):
            for b in (0, 1):
                k = 2 * k2 + b
                b1 = 1 - b

                @pl.when(k >= 1)
                def _():
                    wait_wb(k - 1, b1)

                @pl.when(k + 1 < n_chunks)
                def _():
                    fire_gathers(k + 1, b1)

                wait_gathers(k, b)

                def add_row(i, carry2):
                    for j in range(D // 16):
                        s = pl.ds(j * 16, 16)
                        bufa[b][i, s] = bufa[b][i, s] + bufb[b][i, s]
                    return carry2

                lax.fori_loop(0, c, add_row, 0, unroll=4)
                pltpu.async_copy(
                    bufa[b], out_hbm.at[pl.ds(base + k * c, c)], wb[b])
            return carry

        lax.fori_loop(0, n_chunks // 2, pair, 0)
        wait_wb(n_chunks - 1, 1)

    return pl.kernel(
        body,
        out_type=jax.ShapeDtypeStruct((n, D), jnp.float32),
        mesh=mesh,
        compiler_params=pltpu.CompilerParams(use_tc_tiling_on_sc=False),
        scratch_types=[
            pltpu.VMEM((per_w,), jnp.int32),
            pltpu.VMEM((per_w,), jnp.int32),
            pltpu.VMEM((c, D), jnp.float32),
            pltpu.VMEM((c, D), jnp.float32),
            pltpu.VMEM((c, D), jnp.float32),
            pltpu.VMEM((c, D), jnp.float32),
            pltpu.SemaphoreType.DMA,
            pltpu.SemaphoreType.DMA,
            pltpu.SemaphoreType.DMA,
            pltpu.SemaphoreType.DMA,
            pltpu.SemaphoreType.DMA,
            pltpu.SemaphoreType.DMA,
        ],
    )(ix, iy, wx, wy)


def kernel(x_coord, Wx, Wy):
    b, l, _ = x_coord.shape
    n = b * l
    ix = x_coord[..., 0].reshape(n)
    iy = x_coord[..., 1].reshape(n)
    out = _lookup_sum(ix, iy, Wx, Wy, n, 128)
    return out.reshape(b, l, D)


# speedup vs baseline: 1.9208x; 1.0048x over previous
import functools

import jax
import jax.numpy as jnp
from jax import lax
from jax.experimental import pallas as pl
from jax.experimental.pallas import tpu as pltpu
from jax.experimental.pallas import tpu_sc as plsc

D = 64
NC, NS = 2, 16
NW = NC * NS


@functools.partial(jax.jit, static_argnums=(4, 5))
def _lookup_sum(ix, iy, wx, wy, n, c):
    per_w = n // NW
    n_chunks = per_w // c
    assert n_chunks % 2 == 0
    mesh = plsc.VectorSubcoreMesh(core_axis_name="c", subcore_axis_name="s")

    def body(ix_hbm, iy_hbm, wx_hbm, wy_hbm, out_hbm,
             idxx_v, idxy_v, ba0, ba1, bb0, bb1,
             ga0, ga1, gb0, gb1, wb0, wb1):
        wid = lax.axis_index("s") * NC + lax.axis_index("c")
        base = wid * per_w
        bufa = (ba0, ba1)
        bufb = (bb0, bb1)
        ga = (ga0, ga1)
        gb = (gb0, gb1)
        wb = (wb0, wb1)

        pltpu.sync_copy(ix_hbm.at[pl.ds(base, per_w)], idxx_v)
        pltpu.sync_copy(iy_hbm.at[pl.ds(base, per_w)], idxy_v)

        def fire_gathers(k, b):
            s = pl.ds(k * c, c)
            pltpu.async_copy(wx_hbm.at[idxx_v.at[s]], bufa[b], ga[b])
            pltpu.async_copy(wy_hbm.at[idxy_v.at[s]], bufb[b], gb[b])

        def wait_gathers(k, b):
            s = pl.ds(k * c, c)
            pltpu.make_async_copy(wx_hbm.at[idxx_v.at[s]], bufa[b], ga[b]).wait()
            pltpu.make_async_copy(wy_hbm.at[idxy_v.at[s]], bufb[b], gb[b]).wait()

        def wait_wb(k, b):
            pltpu.make_async_copy(
                bufa[b], out_hbm.at[pl.ds(base + k * c, c)], wb[b]).wait()

        fire_gathers(0, 0)

        def pair(k2, carry):
            for b in (0, 1):
                k = 2 * k2 + b
                b1 = 1 - b

                @pl.when(k >= 1)
                def _():
                    wait_wb(k - 1, b1)

                @pl.when(k + 1 < n_chunks)
                def _():
                    fire_gathers(k + 1, b1)

                wait_gathers(k, b)

                def add_row(i, carry2):
                    for j in range(D // 16):
                        s = pl.ds(j * 16, 16)
                        bufa[b][i, s] = bufa[b][i, s] + bufb[b][i, s]
                    return carry2

                lax.fori_loop(0, c, add_row, 0, unroll=4)
                pltpu.async_copy(
                    bufa[b], out_hbm.at[pl.ds(base + k * c, c)], wb[b])
            return carry

        lax.fori_loop(0, n_chunks // 2, pair, 0)
        wait_wb(n_chunks - 1, 1)

    return pl.kernel(
        body,
        out_type=jax.ShapeDtypeStruct((n, D), jnp.float32),
        mesh=mesh,
        compiler_params=pltpu.CompilerParams(use_tc_tiling_on_sc=False),
        scratch_types=[
            pltpu.VMEM((per_w,), jnp.int32),
            pltpu.VMEM((per_w,), jnp.int32),
            pltpu.VMEM((c, D), jnp.float32),
            pltpu.VMEM((c, D), jnp.float32),
            pltpu.VMEM((c, D), jnp.float32),
            pltpu.VMEM((c, D), jnp.float32),
            pltpu.SemaphoreType.DMA,
            pltpu.SemaphoreType.DMA,
            pltpu.SemaphoreType.DMA,
            pltpu.SemaphoreType.DMA,
            pltpu.SemaphoreType.DMA,
            pltpu.SemaphoreType.DMA,
        ],
    )(ix, iy, wx, wy)


def kernel(x_coord, Wx, Wy):
    b, l, _ = x_coord.shape
    n = b * l
    ix = x_coord[..., 0].reshape(n)
    iy = x_coord[..., 1].reshape(n)
    out = _lookup_sum(ix, iy, Wx, Wy, n, 256)
    return out.reshape(b, l, D)
